# Initial kernel scaffold; baseline (speedup 1.0000x reference)
import jax
import jax.numpy as jnp
from jax.experimental import pallas as pl


def kernel(embed_init, triplets, weights):
    raise NotImplementedError("write your pallas kernel here")


# speedup vs baseline: 2.1715x; 2.1715x over previous
import jax
import jax.numpy as jnp
from jax import lax
from jax.experimental import pallas as pl
from jax.experimental.pallas import tpu as pltpu
from jax.experimental.pallas import tpu_sc as plsc

N_ROWS = 100_000
T_TRI = 1_000_000
L = 16
NC, NS = 2, 16
NW = NC * NS
CPT = 32_768
BLK = 2_048
GPB = BLK // L
NB = CPT // BLK
TAIL_TILE = 30
TAIL_START = 30 * CPT + 8 * BLK
TAIL_N = T_TRI - TAIL_START
HI_MASK = -65536


def _pack_table(embed):
    b16 = embed.astype(jnp.bfloat16)
    u16 = jax.lax.bitcast_convert_type(b16, jnp.uint16).astype(jnp.uint32)
    word = u16[:, 0] | (u16[:, 1] << 16)
    return jax.lax.bitcast_convert_type(word, jnp.int32)


def _body(tab_hbm, trip_hbm, wt_hbm, loss_hbm, viol_hbm,
          tab_v, tbuf, wbuf, st_l, st_v):
    c = lax.axis_index("c")
    s = lax.axis_index("s")
    w = s * NC + c
    pltpu.sync_copy(tab_hbm, tab_v)
    io3 = lax.iota(jnp.int32, L) * 3

    def group(g, carry, al_av):
        al, av = al_av
        p = io3 + g * (3 * L)
        ii = plsc.load_gather(tbuf, [p])
        jj = plsc.load_gather(tbuf, [p + 1])
        kk = plsc.load_gather(tbuf, [p + 2])
        wi = plsc.load_gather(tab_v, [ii])
        wj = plsc.load_gather(tab_v, [jj])
        wk = plsc.load_gather(tab_v, [kk])
        xi = plsc.bitcast(wi << 16, jnp.float32)
        yi = plsc.bitcast(wi & HI_MASK, jnp.float32)
        xj = plsc.bitcast(wj << 16, jnp.float32)
        yj = plsc.bitcast(wj & HI_MASK, jnp.float32)
        xk = plsc.bitcast(wk << 16, jnp.float32)
        yk = plsc.bitcast(wk & HI_MASK, jnp.float32)
        dx1 = xi - xj
        dy1 = yi - yj
        dx2 = xi - xk
        dy2 = yi - yk
        dij = 1.0 + dx1 * dx1 + dy1 * dy1
        dik = 1.0 + dx2 * dx2 + dy2 * dy2
        ww = wbuf[pl.ds(g * L, L)]
        al = al + ww * dij / (dij + dik)
        av = av + jnp.where(dij > dik, 1.0, 0.0)
        return al, av

    base = w * CPT
    nb = jnp.clip((T_TRI - base) // BLK, 0, NB)

    def block(b, al_av):
        start = base + b * BLK
        pltpu.sync_copy(trip_hbm.at[pl.ds(start * 3, BLK * 3)], tbuf)
        pltpu.sync_copy(wt_hbm.at[pl.ds(start, BLK)], wbuf)
        return lax.fori_loop(0, GPB, lambda g, cr: group(g, None, cr),
                             al_av, unroll=4)

    zero = jnp.zeros((L,), jnp.float32)
    al, av = lax.fori_loop(0, nb, block, (zero, zero))
    st_l[...] = al
    st_v[...] = av

    @pl.when(w == TAIL_TILE)
    def _tail():
        pltpu.sync_copy(trip_hbm.at[pl.ds(TAIL_START * 3, TAIL_N * 3)],
                        tbuf.at[pl.ds(0, TAIL_N * 3)])
        pltpu.sync_copy(wt_hbm.at[pl.ds(TAIL_START, TAIL_N)],
                        wbuf.at[pl.ds(0, TAIL_N)])
        tl, tv = lax.fori_loop(0, TAIL_N // L,
                               lambda g, cr: group(g, None, cr),
                               (st_l[...], st_v[...]))
        st_l[...] = tl
        st_v[...] = tv

    pltpu.sync_copy(st_l, loss_hbm.at[w])
    pltpu.sync_copy(st_v, viol_hbm.at[w])


def kernel(embed_init, triplets, weights):
    tab = _pack_table(embed_init)
    trip_flat = triplets.astype(jnp.int32).reshape(-1)
    mesh = plsc.VectorSubcoreMesh(core_axis_name="c", subcore_axis_name="s",
                                  num_cores=NC, num_subcores=NS)
    fn = pl.kernel(
        _body,
        out_type=(jax.ShapeDtypeStruct((NW, L), jnp.float32),
                  jax.ShapeDtypeStruct((NW, L), jnp.float32)),
        mesh=mesh,
        compiler_params=pltpu.CompilerParams(needs_layout_passes=False),
        scratch_types=[
            pltpu.VMEM((N_ROWS,), jnp.int32),
            pltpu.VMEM((BLK * 3,), jnp.int32),
            pltpu.VMEM((BLK,), jnp.float32),
            pltpu.VMEM((L,), jnp.float32),
            pltpu.VMEM((L,), jnp.float32),
        ],
    )
    loss_p, viol_p = fn(tab, trip_flat, weights)
    return jnp.sum(loss_p), jnp.sum(viol_p)


# speedup vs baseline: 51.8735x; 23.8879x over previous
import jax
import jax.numpy as jnp
from jax import lax
from jax.experimental import pallas as pl
from jax.experimental.pallas import tpu as pltpu
from jax.experimental.pallas import tpu_sc as plsc

N_ROWS = 100_000
T_TRI = 1_000_000
L = 16
NC, NS = 2, 16
NW = NC * NS
CPT = 32_768
BLK = 2_048
GPB = BLK // L
NB = CPT // BLK
TAIL_TILE = 30
TAIL_START = 30 * CPT + 8 * BLK
TAIL_N = T_TRI - TAIL_START
HI_MASK = -65536


def _pack_table(embed):
    b16 = embed.astype(jnp.bfloat16)
    u16 = jax.lax.bitcast_convert_type(b16, jnp.uint16).astype(jnp.uint32)
    word = u16[:, 0] | (u16[:, 1] << 16)
    return jax.lax.bitcast_convert_type(word, jnp.int32)


def _body(tab_hbm, i_hbm, j_hbm, k_hbm, wt_hbm, loss_hbm, viol_hbm,
          tab_v, ibuf, jbuf, kbuf, wbuf, st_l, st_v):
    c = lax.axis_index("c")
    s = lax.axis_index("s")
    w = s * NC + c
    pltpu.sync_copy(tab_hbm, tab_v)

    def group(g, al_av):
        al, av = al_av
        ii = ibuf[pl.ds(g * L, L)]
        jj = jbuf[pl.ds(g * L, L)]
        kk = kbuf[pl.ds(g * L, L)]
        wi = plsc.load_gather(tab_v, [ii])
        wj = plsc.load_gather(tab_v, [jj])
        wk = plsc.load_gather(tab_v, [kk])
        xi = plsc.bitcast(wi << 16, jnp.float32)
        yi = plsc.bitcast(wi & HI_MASK, jnp.float32)
        xj = plsc.bitcast(wj << 16, jnp.float32)
        yj = plsc.bitcast(wj & HI_MASK, jnp.float32)
        xk = plsc.bitcast(wk << 16, jnp.float32)
        yk = plsc.bitcast(wk & HI_MASK, jnp.float32)
        dx1 = xi - xj
        dy1 = yi - yj
        dx2 = xi - xk
        dy2 = yi - yk
        dij = 1.0 + dx1 * dx1 + dy1 * dy1
        dik = 1.0 + dx2 * dx2 + dy2 * dy2
        ww = wbuf[pl.ds(g * L, L)]
        al = al + ww * dij / (dij + dik)
        av = av + jnp.where(dij > dik, 1.0, 0.0)
        return al, av

    base = w * CPT
    nb = jnp.clip((T_TRI - base) // BLK, 0, NB)

    def block(b, al_av):
        start = base + b * BLK
        pltpu.sync_copy(i_hbm.at[pl.ds(start, BLK)], ibuf)
        pltpu.sync_copy(j_hbm.at[pl.ds(start, BLK)], jbuf)
        pltpu.sync_copy(k_hbm.at[pl.ds(start, BLK)], kbuf)
        pltpu.sync_copy(wt_hbm.at[pl.ds(start, BLK)], wbuf)
        return lax.fori_loop(0, GPB, group, al_av, unroll=4)

    zero = jnp.zeros((L,), jnp.float32)
    al, av = lax.fori_loop(0, nb, block, (zero, zero))
    st_l[...] = al
    st_v[...] = av

    @pl.when(w == TAIL_TILE)
    def _tail():
        pltpu.sync_copy(i_hbm.at[pl.ds(TAIL_START, TAIL_N)],
                        ibuf.at[pl.ds(0, TAIL_N)])
        pltpu.sync_copy(j_hbm.at[pl.ds(TAIL_START, TAIL_N)],
                        jbuf.at[pl.ds(0, TAIL_N)])
        pltpu.sync_copy(k_hbm.at[pl.ds(TAIL_START, TAIL_N)],
                        kbuf.at[pl.ds(0, TAIL_N)])
        pltpu.sync_copy(wt_hbm.at[pl.ds(TAIL_START, TAIL_N)],
                        wbuf.at[pl.ds(0, TAIL_N)])
        tl, tv = lax.fori_loop(0, TAIL_N // L, group, (st_l[...], st_v[...]))
        st_l[...] = tl
        st_v[...] = tv

    pltpu.sync_copy(st_l, loss_hbm.at[w])
    pltpu.sync_copy(st_v, viol_hbm.at[w])


def kernel(embed_init, triplets, weights):
    tab = _pack_table(embed_init)
    trip = triplets.astype(jnp.int32)
    i_idx = trip[:, 0]
    j_idx = trip[:, 1]
    k_idx = trip[:, 2]
    mesh = plsc.VectorSubcoreMesh(core_axis_name="c", subcore_axis_name="s",
                                  num_cores=NC, num_subcores=NS)
    fn = pl.kernel(
        _body,
        out_type=(jax.ShapeDtypeStruct((NW, L), jnp.float32),
                  jax.ShapeDtypeStruct((NW, L), jnp.float32)),
        mesh=mesh,
        compiler_params=pltpu.CompilerParams(needs_layout_passes=False),
        scratch_types=[
            pltpu.VMEM((N_ROWS,), jnp.int32),
            pltpu.VMEM((BLK,), jnp.int32),
            pltpu.VMEM((BLK,), jnp.int32),
            pltpu.VMEM((BLK,), jnp.int32),
            pltpu.VMEM((BLK,), jnp.float32),
            pltpu.VMEM((L,), jnp.float32),
            pltpu.VMEM((L,), jnp.float32),
        ],
    )
    loss_p, viol_p = fn(tab, i_idx, j_idx, k_idx, weights)
    return jnp.sum(loss_p), jnp.sum(viol_p)


# speedup vs baseline: 55.0802x; 1.0618x over previous
import jax
import jax.numpy as jnp
from jax import lax
from jax.experimental import pallas as pl
from jax.experimental.pallas import tpu as pltpu
from jax.experimental.pallas import tpu_sc as plsc

N_ROWS = 100_000
T_TRI = 1_000_000
L = 16
NC, NS = 2, 16
NW = NC * NS
CPT = 32_768
BLK = 2_048
GPB = BLK // L
NB = CPT // BLK
CHUNK = 512
CW = BLK * 4
N_CHUNKS = (T_TRI + 127) // 128 + 0
PAD_ROWS = N_CHUNKS * 128 - T_TRI
TAIL_TILE = 30
TAIL_START = 30 * CPT + 8 * BLK
TAIL_N = T_TRI - TAIL_START
TAIL_CW = 5 * CHUNK
HI_MASK = -65536


def _pack_table(embed):
    b16 = embed.astype(jnp.bfloat16)
    u16 = jax.lax.bitcast_convert_type(b16, jnp.uint16).astype(jnp.uint32)
    word = u16[:, 0] | (u16[:, 1] << 16)
    return jax.lax.bitcast_convert_type(word, jnp.int32)


def _body(tab_hbm, trip_hbm, wt_hbm, loss_hbm, viol_hbm,
          tab_v, cb0, cb1, wb0, wb1, st_l, st_v, sem0, sem1):
    c = lax.axis_index("c")
    s = lax.axis_index("s")
    w = s * NC + c
    base = w * CPT
    nb = jnp.clip((T_TRI - base) // BLK, 0, NB)

    def start_block(b, cb, wb, sem):
        st = base + b * BLK
        pltpu.make_async_copy(trip_hbm.at[pl.ds(st * 4, CW)], cb, sem).start()
        pltpu.make_async_copy(wt_hbm.at[pl.ds(st, BLK)], wb, sem).start()

    def wait_block(b, cb, wb, sem):
        st = base + b * BLK
        pltpu.make_async_copy(trip_hbm.at[pl.ds(st * 4, CW)], cb, sem).wait()
        pltpu.make_async_copy(wt_hbm.at[pl.ds(st, BLK)], wb, sem).wait()

    @pl.when(nb > 0)
    def _prime():
        start_block(0, cb0, wb0, sem0)
        start_block(1, cb1, wb1, sem1)

    pltpu.sync_copy(tab_hbm, tab_v)

    def group(g, al_av, cb, wb):
        al, av = al_av
        q = (g // 8) * CHUNK + (g % 8) * L
        ii = cb[pl.ds(q, L)]
        jj = cb[pl.ds(q + 128, L)]
        kk = cb[pl.ds(q + 256, L)]
        wi = plsc.load_gather(tab_v, [ii])
        wj = plsc.load_gather(tab_v, [jj])
        wk = plsc.load_gather(tab_v, [kk])
        xi = plsc.bitcast(wi << 16, jnp.float32)
        yi = plsc.bitcast(wi & HI_MASK, jnp.float32)
        xj = plsc.bitcast(wj << 16, jnp.float32)
        yj = plsc.bitcast(wj & HI_MASK, jnp.float32)
        xk = plsc.bitcast(wk << 16, jnp.float32)
        yk = plsc.bitcast(wk & HI_MASK, jnp.float32)
        dx1 = xi - xj
        dy1 = yi - yj
        dx2 = xi - xk
        dy2 = yi - yk
        dij = 1.0 + dx1 * dx1 + dy1 * dy1
        dik = 1.0 + dx2 * dx2 + dy2 * dy2
        ww = wb[pl.ds(g * L, L)]
        al = al + ww * dij / (dij + dik)
        av = av + jnp.where(dij > dik, 1.0, 0.0)
        return al, av

    def pair(i, al_av):
        b0 = 2 * i
        wait_block(b0, cb0, wb0, sem0)

        @pl.when(b0 + 2 < nb)
        def _():
            start_block(b0 + 2, cb0, wb0, sem0)

        al_av = lax.fori_loop(0, GPB, lambda g, cr: group(g, cr, cb0, wb0),
                              al_av, unroll=8)
        wait_block(b0 + 1, cb1, wb1, sem1)

        @pl.when(b0 + 3 < nb)
        def _():
            start_block(b0 + 3, cb1, wb1, sem1)

        return lax.fori_loop(0, GPB, lambda g, cr: group(g, cr, cb1, wb1),
                             al_av, unroll=8)

    zero = jnp.zeros((L,), jnp.float32)
    al, av = lax.fori_loop(0, nb // 2, pair, (zero, zero))
    st_l[...] = al
    st_v[...] = av

    @pl.when(w == TAIL_TILE)
    def _tail():
        pltpu.sync_copy(trip_hbm.at[pl.ds(TAIL_START * 4, TAIL_CW)],
                        cb0.at[pl.ds(0, TAIL_CW)])
        pltpu.sync_copy(wt_hbm.at[pl.ds(TAIL_START, TAIL_N)],
                        wb0.at[pl.ds(0, TAIL_N)])
        tl, tv = lax.fori_loop(0, TAIL_N // L,
                               lambda g, cr: group(g, cr, cb0, wb0),
                               (st_l[...], st_v[...]))
        st_l[...] = tl
        st_v[...] = tv

    pltpu.sync_copy(st_l, loss_hbm.at[w])
    pltpu.sync_copy(st_v, viol_hbm.at[w])


def kernel(embed_init, triplets, weights):
    tab = _pack_table(embed_init)
    trip = triplets.astype(jnp.int32)
    padded = jnp.pad(trip, ((0, PAD_ROWS), (0, 1)))
    trip4 = padded.reshape(N_CHUNKS, 128, 4).transpose(0, 2, 1).reshape(-1)
    mesh = plsc.VectorSubcoreMesh(core_axis_name="c", subcore_axis_name="s",
                                  num_cores=NC, num_subcores=NS)
    fn = pl.kernel(
        _body,
        out_type=(jax.ShapeDtypeStruct((NW, L), jnp.float32),
                  jax.ShapeDtypeStruct((NW, L), jnp.float32)),
        mesh=mesh,
        compiler_params=pltpu.CompilerParams(needs_layout_passes=False),
        scratch_types=[
            pltpu.VMEM((N_ROWS,), jnp.int32),
            pltpu.VMEM((CW,), jnp.int32),
            pltpu.VMEM((CW,), jnp.int32),
            pltpu.VMEM((BLK,), jnp.float32),
            pltpu.VMEM((BLK,), jnp.float32),
            pltpu.VMEM((L,), jnp.float32),
            pltpu.VMEM((L,), jnp.float32),
            pltpu.SemaphoreType.DMA,
            pltpu.SemaphoreType.DMA,
        ],
    )
    loss_p, viol_p = fn(tab, trip4, weights)
    return jnp.sum(loss_p), jnp.sum(viol_p)


# speedup vs baseline: 86.7426x; 1.5748x over previous
import jax
import jax.numpy as jnp
from jax import lax
from jax.experimental import pallas as pl
from jax.experimental.pallas import tpu as pltpu
from jax.experimental.pallas import tpu_sc as plsc

N_ROWS = 100_000
T_TRI = 1_000_000
L = 16
NC, NS = 2, 16
NW = NC * NS
CPT = 32_768
BLK = 2_048
GPB = BLK // L
NB = CPT // BLK
TAIL_TILE = 30
TAIL_START = 30 * CPT + 8 * BLK
TAIL_N = T_TRI - TAIL_START
HI_MASK = -65536


def _pack_table(embed):
    b16 = embed.astype(jnp.bfloat16)
    u16 = jax.lax.bitcast_convert_type(b16, jnp.uint16).astype(jnp.uint32)
    word = u16[:, 0] | (u16[:, 1] << 16)
    return jax.lax.bitcast_convert_type(word, jnp.int32)


def _body(tab_hbm, trip_hbm, wt_hbm, loss_hbm, viol_hbm,
          tab_v, ib0, ib1, jb0, jb1, kb0, kb1, wb0, wb1,
          st_l, st_v, sem0, sem1):
    c = lax.axis_index("c")
    s = lax.axis_index("s")
    w = s * NC + c
    base = w * CPT
    nb = jnp.clip((T_TRI - base) // BLK, 0, NB)

    def copies(b, ib, jb, kb, wb, sem):
        st = base + b * BLK
        return (
            pltpu.make_async_copy(trip_hbm.at[pl.ds(st, BLK)], ib, sem),
            pltpu.make_async_copy(trip_hbm.at[pl.ds(T_TRI + st, BLK)], jb, sem),
            pltpu.make_async_copy(trip_hbm.at[pl.ds(2 * T_TRI + st, BLK)], kb, sem),
            pltpu.make_async_copy(wt_hbm.at[pl.ds(st, BLK)], wb, sem),
        )

    def start_block(b, ib, jb, kb, wb, sem):
        for cp in copies(b, ib, jb, kb, wb, sem):
            cp.start()

    def wait_block(b, ib, jb, kb, wb, sem):
        for cp in copies(b, ib, jb, kb, wb, sem):
            cp.wait()

    @pl.when(nb > 0)
    def _prime():
        start_block(0, ib0, jb0, kb0, wb0, sem0)
        start_block(1, ib1, jb1, kb1, wb1, sem1)

    pltpu.sync_copy(tab_hbm, tab_v)

    def group(g, al_av, ib, jb, kb, wb):
        al, av = al_av
        ii = ib[pl.ds(g * L, L)]
        jj = jb[pl.ds(g * L, L)]
        kk = kb[pl.ds(g * L, L)]
        wi = plsc.load_gather(tab_v, [ii])
        wj = plsc.load_gather(tab_v, [jj])
        wk = plsc.load_gather(tab_v, [kk])
        xi = plsc.bitcast(wi << 16, jnp.float32)
        yi = plsc.bitcast(wi & HI_MASK, jnp.float32)
        xj = plsc.bitcast(wj << 16, jnp.float32)
        yj = plsc.bitcast(wj & HI_MASK, jnp.float32)
        xk = plsc.bitcast(wk << 16, jnp.float32)
        yk = plsc.bitcast(wk & HI_MASK, jnp.float32)
        dx1 = xi - xj
        dy1 = yi - yj
        dx2 = xi - xk
        dy2 = yi - yk
        dij = 1.0 + dx1 * dx1 + dy1 * dy1
        dik = 1.0 + dx2 * dx2 + dy2 * dy2
        ww = wb[pl.ds(g * L, L)]
        al = al + ww * dij / (dij + dik)
        av = av + jnp.where(dij > dik, 1.0, 0.0)
        return al, av

    def pair(i, al_av):
        b0 = 2 * i
        wait_block(b0, ib0, jb0, kb0, wb0, sem0)

        @pl.when(b0 + 2 < nb)
        def _():
            start_block(b0 + 2, ib0, jb0, kb0, wb0, sem0)

        al_av = lax.fori_loop(
            0, GPB, lambda g, cr: group(g, cr, ib0, jb0, kb0, wb0),
            al_av, unroll=8)
        wait_block(b0 + 1, ib1, jb1, kb1, wb1, sem1)

        @pl.when(b0 + 3 < nb)
        def _():
            start_block(b0 + 3, ib1, jb1, kb1, wb1, sem1)

        return lax.fori_loop(
            0, GPB, lambda g, cr: group(g, cr, ib1, jb1, kb1, wb1),
            al_av, unroll=8)

    zero = jnp.zeros((L,), jnp.float32)
    al, av = lax.fori_loop(0, nb // 2, pair, (zero, zero))
    st_l[...] = al
    st_v[...] = av

    @pl.when(w == TAIL_TILE)
    def _tail():
        pltpu.sync_copy(trip_hbm.at[pl.ds(TAIL_START, TAIL_N)],
                        ib0.at[pl.ds(0, TAIL_N)])
        pltpu.sync_copy(trip_hbm.at[pl.ds(T_TRI + TAIL_START, TAIL_N)],
                        jb0.at[pl.ds(0, TAIL_N)])
        pltpu.sync_copy(trip_hbm.at[pl.ds(2 * T_TRI + TAIL_START, TAIL_N)],
                        kb0.at[pl.ds(0, TAIL_N)])
        pltpu.sync_copy(wt_hbm.at[pl.ds(TAIL_START, TAIL_N)],
                        wb0.at[pl.ds(0, TAIL_N)])
        tl, tv = lax.fori_loop(
            0, TAIL_N // L, lambda g, cr: group(g, cr, ib0, jb0, kb0, wb0),
            (st_l[...], st_v[...]))
        st_l[...] = tl
        st_v[...] = tv

    pltpu.sync_copy(st_l, loss_hbm.at[w])
    pltpu.sync_copy(st_v, viol_hbm.at[w])


def kernel(embed_init, triplets, weights):
    tab = _pack_table(embed_init)
    trip_t = triplets.astype(jnp.int32).T.reshape(-1)
    mesh = plsc.VectorSubcoreMesh(core_axis_name="c", subcore_axis_name="s",
                                  num_cores=NC, num_subcores=NS)
    blk_i32 = pltpu.VMEM((BLK,), jnp.int32)
    blk_f32 = pltpu.VMEM((BLK,), jnp.float32)
    fn = pl.kernel(
        _body,
        out_type=(jax.ShapeDtypeStruct((NW, L), jnp.float32),
                  jax.ShapeDtypeStruct((NW, L), jnp.float32)),
        mesh=mesh,
        compiler_params=pltpu.CompilerParams(needs_layout_passes=False),
        scratch_types=[
            pltpu.VMEM((N_ROWS,), jnp.int32),
            blk_i32, blk_i32, blk_i32, blk_i32, blk_i32, blk_i32,
            blk_f32, blk_f32,
            pltpu.VMEM((L,), jnp.float32),
            pltpu.VMEM((L,), jnp.float32),
            pltpu.SemaphoreType.DMA,
            pltpu.SemaphoreType.DMA,
        ],
    )
    loss_p, viol_p = fn(tab, trip_t, weights)
    return jnp.sum(loss_p), jnp.sum(viol_p)
